# baseline (speedup 1.0000x reference)
import functools

import jax
import jax.numpy as jnp
from jax import lax
from jax.experimental import pallas as pl
from jax.experimental.pallas import tpu as pltpu, tpu_sc as plsc

_ROWS = 100000
_NW = 32
_RPW = _ROWS // _NW
_CHUNK = 125
_NCHUNK = _RPW // _CHUNK

_SLICES = (
    (0, 0, 64, 0),
    (128, 64, 96, 0),
    (320, 160, 80, 0),
    (64, 0, 64, 1),
    (224, 64, 96, 1),
    (400, 160, 80, 1),
)

_mesh = plsc.VectorSubcoreMesh(core_axis_name="c", subcore_axis_name="s")


@functools.partial(
    pl.kernel,
    mesh=_mesh,
    out_type=(
        jax.ShapeDtypeStruct((_ROWS, 240), jnp.float32),
        jax.ShapeDtypeStruct((_ROWS, 240), jnp.float32),
    ),
    scratch_types=[
        pltpu.VMEM((_CHUNK, w), jnp.float32) for _, _, w, _ in _SLICES
    ] + [pltpu.SemaphoreType.DMA],
    compiler_params=pltpu.CompilerParams(use_tc_tiling_on_sc=False),
)
def _half_split(x_hbm, out0_hbm, out1_hbm, b0, b1, b2, b3, b4, b5, sem):
    wid = lax.axis_index("s") * 2 + lax.axis_index("c")
    base = wid * _RPW
    bufs = (b0, b1, b2, b3, b4, b5)
    outs = (out0_hbm, out1_hbm)

    def chunk(g, carry):
        r0 = base + g * _CHUNK
        copies = []
        for buf, (src_col, _, width, _) in zip(bufs, _SLICES):
            copies.append(pltpu.async_copy(
                x_hbm.at[pl.ds(r0, _CHUNK), pl.ds(src_col, width)],
                buf,
                sem,
            ))
        for c in copies:
            c.wait()
        copies = []
        for buf, (_, dst_col, width, oi) in zip(bufs, _SLICES):
            copies.append(pltpu.async_copy(
                buf,
                outs[oi].at[pl.ds(r0, _CHUNK), pl.ds(dst_col, width)],
                sem,
            ))
        for c in copies:
            c.wait()
        return carry

    lax.fori_loop(0, _NCHUNK, chunk, 0)


def kernel(x):
    return _half_split(x)
